# baseline (speedup 1.0000x reference)
import jax
import jax.numpy as jnp
from jax import lax
from jax.experimental import pallas as pl
from jax.experimental.pallas import tpu as pltpu
from jax.experimental.pallas import tpu_sc as plsc

H = 256
HH = 128
NP = 10240
MB = 2048
NT = 16
NE = 160000
EC = 64
NCH = 160
EPT = EC * NCH
NEP = EPT * NT
TRASH = NP
SPAD = 10496


def _mlp_body(a_ref, h_ref, agg_ref, w1_ref, c1_ref, w2_ref, c2_ref, o_ref):
    hcat = jnp.concatenate([h_ref[0], h_ref[1]], axis=1)
    acat = jnp.concatenate([agg_ref[0], agg_ref[1]], axis=1)
    z = a_ref[0] * hcat + acat
    u = jnp.dot(z, w1_ref[...], preferred_element_type=jnp.float32) + c1_ref[...]
    u = jnp.maximum(u, 0.0)
    v = jnp.dot(u, w2_ref[...], preferred_element_type=jnp.float32) + c2_ref[...]
    v = jnp.maximum(v, 0.0)
    o_ref[0] = v[:, :HH]
    o_ref[1] = v[:, HH:]


@jax.jit
def _mlp(a, h2, agg2, w1, c1, w2, c2):
    return pl.pallas_call(
        _mlp_body,
        grid=(NP // MB,),
        in_specs=[
            pl.BlockSpec(memory_space=pltpu.SMEM),
            pl.BlockSpec((2, MB, HH), lambda i: (0, i, 0)),
            pl.BlockSpec((2, MB, HH), lambda i: (0, i, 0)),
            pl.BlockSpec((H, 2 * H), lambda i: (0, 0)),
            pl.BlockSpec((1, 2 * H), lambda i: (0, 0)),
            pl.BlockSpec((2 * H, H), lambda i: (0, 0)),
            pl.BlockSpec((1, H), lambda i: (0, 0)),
        ],
        out_specs=pl.BlockSpec((2, MB, HH), lambda i: (0, i, 0)),
        out_shape=jax.ShapeDtypeStruct((2, NP, HH), jnp.float32),
    )(a, h2, agg2, w1, c1, w2, c2)


def _edge_body(h2, idxp, t01, t23, out, aggs, hbuf, acc2d, sbuf,
               idxb, idxv, tb0, tb1, sidx, sem):
    c = lax.axis_index("c")
    s = lax.axis_index("s")
    iota = lax.broadcasted_iota(jnp.int32, (16,), 0)
    zf = jnp.zeros((16,), jnp.float32)

    def _zq(q, _):
        acc2d[0, pl.ds(q * 16, 16)] = zf
        return 0
    lax.fori_loop(0, 8, _zq, 0)

    def _zs(i, _):
        def _zq2(q, _):
            sbuf[i, pl.ds(q * 16, 16)] = zf
            return 0
        return lax.fori_loop(0, 8, _zq2, 0)
    lax.fori_loop(0, 16, _zs, 0)

    zb = s * (SPAD // NT)

    def _zcp(i, _):
        pltpu.sync_copy(sbuf, aggs.at[pl.ds(zb + i * 16, 16)])
        return 0
    lax.fori_loop(0, (SPAD // NT) // 16, _zcp, 0)

    pltpu.sync_copy(t01.at[c], tb0)
    pltpu.sync_copy(t23.at[c], tb1)
    plsc.subcore_barrier()

    pltpu.sync_copy(idxp.at[s, 0], idxb.at[0])
    pltpu.sync_copy(idxp.at[s, 0, pl.ds(0, EC)], idxv.at[0])
    pltpu.async_copy(h2.at[c].at[idxv.at[0]], hbuf.at[0], sem)

    def _chunk(g, carry):
        cur = g & 1
        nxt = (g + 1) & 1

        @pl.when(g + 1 < NCH)
        def _pref():
            pltpu.sync_copy(idxp.at[s, g + 1], idxb.at[nxt])
            pltpu.sync_copy(idxp.at[s, g + 1, pl.ds(0, EC)], idxv.at[nxt])
        pltpu.make_async_copy(h2.at[c].at[idxv.at[cur]], hbuf.at[cur], sem).wait()

        @pl.when(g + 1 < NCH)
        def _pref2():
            pltpu.async_copy(h2.at[c].at[idxv.at[nxt]], hbuf.at[nxt], sem)

        def _group(j, gcarry):
            eb = j * 16
            dv = idxb[cur, pl.ds(EC + eb, 16)]
            i0v = idxb[cur, pl.ds(2 * EC + eb, 16)]
            i1v = idxb[cur, pl.ds(3 * EC + eb, 16)]
            cr = gcarry
            for k in range(16):
                dprev, fr, df = cr
                d = dv[k]
                i0 = i0v[k]
                i1 = i1v[k]
                fresh = d != dprev
                real = fresh & (dprev >= 0)

                @pl.when(real & (fr == 1))
                def _first_flush():
                    def _cp(q, _):
                        sbuf[0, pl.ds(q * 16, 16)] = acc2d[0, pl.ds(q * 16, 16)]
                        return 0
                    lax.fori_loop(0, 8, _cp, 0)

                @pl.when(real & (fr == 0))
                def _flush():
                    pltpu.sync_copy(acc2d, aggs.at[pl.ds(dprev, 1)])

                df = jnp.where(real & (fr == 1), dprev, df)
                fr = jnp.where(real & (fr == 1), 0, fr)
                keepv = jnp.full((16,), jnp.where(fresh, 0.0, 1.0))
                ek = eb + k

                def _qloop(q, _):
                    qs = q * 16
                    hv = hbuf[cur, ek, pl.ds(qs, 16)]
                    e0 = tb0[i0, pl.ds(qs, 16)]
                    e1 = tb1[i1, pl.ds(qs, 16)]
                    m = jnp.maximum(hv + e0 + e1, 0.0)
                    a = acc2d[0, pl.ds(qs, 16)]
                    acc2d[0, pl.ds(qs, 16)] = a * keepv + m
                    return 0
                lax.fori_loop(0, 8, _qloop, 0)
                cr = (d, fr, df)
            return cr

        return lax.fori_loop(0, EC // 16, _group, carry)

    dprev, fr, df = lax.fori_loop(
        0, NCH, _chunk, (jnp.int32(-1), jnp.int32(1), jnp.int32(TRASH)))

    df = jnp.where(fr == 1, dprev, df)
    dl = jnp.where(fr == 1, TRASH, dprev)
    slot = jnp.where(fr == 1, 0, 1)

    def _cp(q, _):
        sbuf[slot, pl.ds(q * 16, 16)] = acc2d[0, pl.ds(q * 16, 16)]
        return 0
    lax.fori_loop(0, 8, _cp, 0)

    plsc.subcore_barrier()
    sidxv = jnp.full((16,), TRASH, jnp.int32)
    sidxv = jnp.where(iota == 0, df, sidxv)
    sidxv = jnp.where(iota == 1, dl, sidxv)
    sidx[pl.ds(0, 16)] = sidxv
    pltpu.sync_copy(sbuf, aggs.at[sidx], add=True)
    plsc.subcore_barrier()
    rb = NP // NT
    pltpu.sync_copy(aggs.at[pl.ds(s * rb, rb)], out.at[c, pl.ds(s * rb, rb)])


@jax.jit
def _edge(h2, idxp, t01, t23):
    return pl.kernel(
        _edge_body,
        out_type=jax.ShapeDtypeStruct((2, NP, HH), jnp.float32),
        mesh=plsc.VectorSubcoreMesh(core_axis_name="c", subcore_axis_name="s"),
        scratch_types=[
            pltpu.VMEM_SHARED((SPAD, HH), jnp.float32),
            pltpu.VMEM((2, EC, HH), jnp.float32),
            pltpu.VMEM((1, HH), jnp.float32),
            pltpu.VMEM((16, HH), jnp.float32),
            pltpu.VMEM((2, 6 * EC), jnp.int32),
            pltpu.VMEM((2, EC), jnp.int32),
            pltpu.VMEM((64, HH), jnp.float32),
            pltpu.VMEM((64, HH), jnp.float32),
            pltpu.VMEM((16,), jnp.int32),
            pltpu.SemaphoreType.DMA,
        ],
    )(h2, idxp, t01, t23)


def kernel(x, edge_index, edge_attr, batch, atom_emb, bond_emb, eps, W1, b1, g1, be1, W2, b2, g2, be2, Wp, bp):
    n = x.shape[0]
    src = edge_index[0]
    dst = edge_index[1]

    W1f = W1 * g1[:, None, :]
    c1 = (b1 * g1 + be1)[:, None, :]
    W2f = W2 * g2[:, None, :]
    c2 = (b2 * g2 + be2)[:, None, :]

    h = jnp.zeros((n, H), jnp.float32)
    for f in range(9):
        h = h + atom_emb[f][x[:, f]]
    hp = jnp.zeros((NP, H), jnp.float32).at[:n].set(h)
    h2 = jnp.stack([hp[:, :HH], hp[:, HH:]], 0)

    bt01 = (bond_emb[:, 0, :, None, :] + bond_emb[:, 1, None, :, :]).reshape(4, 64, H)
    bt23 = (bond_emb[:, 2, :, None, :] + bond_emb[:, 3, None, :, :]).reshape(4, 64, H)
    bt01h = jnp.stack([bt01[:, :, :HH], bt01[:, :, HH:]], 1)
    bt23h = jnp.stack([bt23[:, :, :HH], bt23[:, :, HH:]], 1)
    i01 = edge_attr[:, 0] + 8 * edge_attr[:, 1]
    i23 = edge_attr[:, 2] + 8 * edge_attr[:, 3]

    dsts, srcs, i01s, i23s = lax.sort([dst, src, i01, i23], num_keys=1)
    padn = NEP - NE
    dstp = jnp.concatenate([dsts, jnp.full((padn,), TRASH, jnp.int32)])
    srcp = jnp.concatenate([srcs, jnp.zeros((padn,), jnp.int32)])
    i01p = jnp.concatenate([i01s, jnp.zeros((padn,), jnp.int32)])
    i23p = jnp.concatenate([i23s, jnp.zeros((padn,), jnp.int32)])
    z = jnp.zeros((NEP,), jnp.int32)
    idxp = (jnp.stack([srcp, dstp, i01p, i23p, z, z], 0)
            .reshape(6, NT, NCH, EC).transpose(1, 2, 0, 3).reshape(NT, NCH, 6 * EC))

    for i in range(4):
        agg2 = _edge(h2, idxp, bt01h[i], bt23h[i])
        h2 = _mlp(1.0 + eps[i:i + 1], h2, agg2, W1f[i], c1[i], W2f[i], c2[i])

    nr = jnp.concatenate([h2[0], h2[1]], axis=1)[:n]
    sums = jax.ops.segment_sum(nr, batch, num_segments=64)
    cnt = jax.ops.segment_sum(jnp.ones((n,), jnp.float32), batch, num_segments=64)
    hg = sums / jnp.maximum(cnt, 1.0)[:, None]
    return hg @ Wp + bp


# speedup vs baseline: 1.0046x; 1.0046x over previous
import jax
import jax.numpy as jnp
from jax import lax
from jax.experimental import pallas as pl
from jax.experimental.pallas import tpu as pltpu
from jax.experimental.pallas import tpu_sc as plsc

H = 256
HH = 128
NP = 10240
MB = 2048
NT = 16
NE = 160000
EC = 64
NCH = 160
EPT = EC * NCH
NEP = EPT * NT
TRASH = NP
SPAD = 10496


def _mlp_body(a_ref, h_ref, agg_ref, w1_ref, c1_ref, w2_ref, c2_ref, o_ref):
    hcat = jnp.concatenate([h_ref[0], h_ref[1]], axis=1)
    acat = jnp.concatenate([agg_ref[0], agg_ref[1]], axis=1)
    z = a_ref[0] * hcat + acat
    u = jnp.dot(z, w1_ref[...], preferred_element_type=jnp.float32) + c1_ref[...]
    u = jnp.maximum(u, 0.0)
    v = jnp.dot(u, w2_ref[...], preferred_element_type=jnp.float32) + c2_ref[...]
    v = jnp.maximum(v, 0.0)
    o_ref[0] = v[:, :HH]
    o_ref[1] = v[:, HH:]


@jax.jit
def _mlp(a, h2, agg2, w1, c1, w2, c2):
    return pl.pallas_call(
        _mlp_body,
        grid=(NP // MB,),
        in_specs=[
            pl.BlockSpec(memory_space=pltpu.SMEM),
            pl.BlockSpec((2, MB, HH), lambda i: (0, i, 0)),
            pl.BlockSpec((2, MB, HH), lambda i: (0, i, 0)),
            pl.BlockSpec((H, 2 * H), lambda i: (0, 0)),
            pl.BlockSpec((1, 2 * H), lambda i: (0, 0)),
            pl.BlockSpec((2 * H, H), lambda i: (0, 0)),
            pl.BlockSpec((1, H), lambda i: (0, 0)),
        ],
        out_specs=pl.BlockSpec((2, MB, HH), lambda i: (0, i, 0)),
        out_shape=jax.ShapeDtypeStruct((2, NP, HH), jnp.float32),
    )(a, h2, agg2, w1, c1, w2, c2)


def _edge_body(h2, idxp, t01, t23, out, aggs, hbuf, acc2d, sbuf,
               idxb, idxv, tb0, tb1, sidx, sem):
    c = lax.axis_index("c")
    s = lax.axis_index("s")
    iota = lax.broadcasted_iota(jnp.int32, (16,), 0)
    zf = jnp.zeros((16,), jnp.float32)

    def _zq(q, _):
        acc2d[0, pl.ds(q * 16, 16)] = zf
        return 0
    lax.fori_loop(0, 8, _zq, 0)

    def _zs(i, _):
        def _zq2(q, _):
            sbuf[i, pl.ds(q * 16, 16)] = zf
            return 0
        return lax.fori_loop(0, 8, _zq2, 0)
    lax.fori_loop(0, 16, _zs, 0)

    zb = s * (SPAD // NT)

    def _zcp(i, _):
        pltpu.sync_copy(sbuf, aggs.at[pl.ds(zb + i * 16, 16)])
        return 0
    lax.fori_loop(0, (SPAD // NT) // 16, _zcp, 0)

    pltpu.sync_copy(t01.at[c], tb0)
    pltpu.sync_copy(t23.at[c], tb1)
    plsc.subcore_barrier()

    pltpu.sync_copy(idxp.at[s, 0], idxb.at[0])
    pltpu.sync_copy(idxp.at[s, 0, pl.ds(0, EC)], idxv.at[0])
    pltpu.async_copy(h2.at[c].at[idxv.at[0]], hbuf.at[0], sem)

    def _chunk(g, carry):
        cur = g & 1
        nxt = (g + 1) & 1

        @pl.when(g + 1 < NCH)
        def _pref():
            pltpu.sync_copy(idxp.at[s, g + 1], idxb.at[nxt])
            pltpu.sync_copy(idxp.at[s, g + 1, pl.ds(0, EC)], idxv.at[nxt])
        pltpu.make_async_copy(h2.at[c].at[idxv.at[cur]], hbuf.at[cur], sem).wait()

        @pl.when(g + 1 < NCH)
        def _pref2():
            pltpu.async_copy(h2.at[c].at[idxv.at[nxt]], hbuf.at[nxt], sem)

        def _group(j, gcarry):
            eb = j * 16
            dv = idxb[cur, pl.ds(EC + eb, 16)]
            i0v = idxb[cur, pl.ds(2 * EC + eb, 16)]
            i1v = idxb[cur, pl.ds(3 * EC + eb, 16)]
            cr = gcarry
            for k in range(16):
                dprev, fr, df = cr
                d = dv[k]
                i0 = i0v[k]
                i1 = i1v[k]
                fresh = d != dprev
                real = fresh & (dprev >= 0)

                @pl.when(real & (fr == 1))
                def _first_flush():
                    for q in range(8):
                        sbuf[0, pl.ds(q * 16, 16)] = acc2d[0, pl.ds(q * 16, 16)]

                @pl.when(real & (fr == 0))
                def _flush():
                    pltpu.sync_copy(acc2d, aggs.at[pl.ds(dprev, 1)])

                df = jnp.where(real & (fr == 1), dprev, df)
                fr = jnp.where(real & (fr == 1), 0, fr)
                keepv = jnp.full((16,), jnp.where(fresh, 0.0, 1.0))
                ek = eb + k
                for q in range(8):
                    qs = q * 16
                    hv = hbuf[cur, ek, pl.ds(qs, 16)]
                    e0 = tb0[i0, pl.ds(qs, 16)]
                    e1 = tb1[i1, pl.ds(qs, 16)]
                    m = jnp.maximum(hv + e0 + e1, 0.0)
                    a = acc2d[0, pl.ds(qs, 16)]
                    acc2d[0, pl.ds(qs, 16)] = a * keepv + m
                cr = (d, fr, df)
            return cr

        return lax.fori_loop(0, EC // 16, _group, carry)

    dprev, fr, df = lax.fori_loop(
        0, NCH, _chunk, (jnp.int32(-1), jnp.int32(1), jnp.int32(TRASH)))

    df = jnp.where(fr == 1, dprev, df)
    dl = jnp.where(fr == 1, TRASH, dprev)
    slot = jnp.where(fr == 1, 0, 1)

    for q in range(8):
        sbuf[slot, pl.ds(q * 16, 16)] = acc2d[0, pl.ds(q * 16, 16)]

    plsc.subcore_barrier()
    sidxv = jnp.full((16,), TRASH, jnp.int32)
    sidxv = jnp.where(iota == 0, df, sidxv)
    sidxv = jnp.where(iota == 1, dl, sidxv)
    sidx[pl.ds(0, 16)] = sidxv
    pltpu.sync_copy(sbuf, aggs.at[sidx], add=True)
    plsc.subcore_barrier()
    rb = NP // NT
    pltpu.sync_copy(aggs.at[pl.ds(s * rb, rb)], out.at[c, pl.ds(s * rb, rb)])


@jax.jit
def _edge(h2, idxp, t01, t23):
    return pl.kernel(
        _edge_body,
        out_type=jax.ShapeDtypeStruct((2, NP, HH), jnp.float32),
        mesh=plsc.VectorSubcoreMesh(core_axis_name="c", subcore_axis_name="s"),
        scratch_types=[
            pltpu.VMEM_SHARED((SPAD, HH), jnp.float32),
            pltpu.VMEM((2, EC, HH), jnp.float32),
            pltpu.VMEM((1, HH), jnp.float32),
            pltpu.VMEM((16, HH), jnp.float32),
            pltpu.VMEM((2, 6 * EC), jnp.int32),
            pltpu.VMEM((2, EC), jnp.int32),
            pltpu.VMEM((64, HH), jnp.float32),
            pltpu.VMEM((64, HH), jnp.float32),
            pltpu.VMEM((16,), jnp.int32),
            pltpu.SemaphoreType.DMA,
        ],
    )(h2, idxp, t01, t23)


def kernel(x, edge_index, edge_attr, batch, atom_emb, bond_emb, eps, W1, b1, g1, be1, W2, b2, g2, be2, Wp, bp):
    n = x.shape[0]
    src = edge_index[0]
    dst = edge_index[1]

    W1f = W1 * g1[:, None, :]
    c1 = (b1 * g1 + be1)[:, None, :]
    W2f = W2 * g2[:, None, :]
    c2 = (b2 * g2 + be2)[:, None, :]

    h = jnp.zeros((n, H), jnp.float32)
    for f in range(9):
        h = h + atom_emb[f][x[:, f]]
    hp = jnp.zeros((NP, H), jnp.float32).at[:n].set(h)
    h2 = jnp.stack([hp[:, :HH], hp[:, HH:]], 0)

    bt01 = (bond_emb[:, 0, :, None, :] + bond_emb[:, 1, None, :, :]).reshape(4, 64, H)
    bt23 = (bond_emb[:, 2, :, None, :] + bond_emb[:, 3, None, :, :]).reshape(4, 64, H)
    bt01h = jnp.stack([bt01[:, :, :HH], bt01[:, :, HH:]], 1)
    bt23h = jnp.stack([bt23[:, :, :HH], bt23[:, :, HH:]], 1)
    i01 = edge_attr[:, 0] + 8 * edge_attr[:, 1]
    i23 = edge_attr[:, 2] + 8 * edge_attr[:, 3]

    dsts, srcs, i01s, i23s = lax.sort([dst, src, i01, i23], num_keys=1)
    padn = NEP - NE
    dstp = jnp.concatenate([dsts, jnp.full((padn,), TRASH, jnp.int32)])
    srcp = jnp.concatenate([srcs, jnp.zeros((padn,), jnp.int32)])
    i01p = jnp.concatenate([i01s, jnp.zeros((padn,), jnp.int32)])
    i23p = jnp.concatenate([i23s, jnp.zeros((padn,), jnp.int32)])
    z = jnp.zeros((NEP,), jnp.int32)
    idxp = (jnp.stack([srcp, dstp, i01p, i23p, z, z], 0)
            .reshape(6, NT, NCH, EC).transpose(1, 2, 0, 3).reshape(NT, NCH, 6 * EC))

    for i in range(4):
        agg2 = _edge(h2, idxp, bt01h[i], bt23h[i])
        h2 = _mlp(1.0 + eps[i:i + 1], h2, agg2, W1f[i], c1[i], W2f[i], c2[i])

    nr = jnp.concatenate([h2[0], h2[1]], axis=1)[:n]
    sums = jax.ops.segment_sum(nr, batch, num_segments=64)
    cnt = jax.ops.segment_sum(jnp.ones((n,), jnp.float32), batch, num_segments=64)
    hg = sums / jnp.maximum(cnt, 1.0)[:, None]
    return hg @ Wp + bp


# speedup vs baseline: 1.1751x; 1.1697x over previous
import jax
import jax.numpy as jnp
from jax import lax
from jax.experimental import pallas as pl
from jax.experimental.pallas import tpu as pltpu
from jax.experimental.pallas import tpu_sc as plsc

H = 256
HH = 128
NP = 10240
MB = 2048
NT = 16
NE = 160000
EC = 64
NCH = 160
EPT = EC * NCH
NEP = EPT * NT
TRASH = NP
SPAD = 10496


def _mlp_body(a_ref, h_ref, agg_ref, w1_ref, c1_ref, w2_ref, c2_ref, o_ref):
    hcat = jnp.concatenate([h_ref[0], h_ref[1]], axis=1)
    acat = jnp.concatenate([agg_ref[0], agg_ref[1]], axis=1)
    z = a_ref[0] * hcat + acat
    u = jnp.dot(z, w1_ref[...], preferred_element_type=jnp.float32) + c1_ref[...]
    u = jnp.maximum(u, 0.0)
    v = jnp.dot(u, w2_ref[...], preferred_element_type=jnp.float32) + c2_ref[...]
    v = jnp.maximum(v, 0.0)
    o_ref[0] = v[:, :HH]
    o_ref[1] = v[:, HH:]


@jax.jit
def _mlp(a, h2, agg2, w1, c1, w2, c2):
    return pl.pallas_call(
        _mlp_body,
        grid=(NP // MB,),
        in_specs=[
            pl.BlockSpec(memory_space=pltpu.SMEM),
            pl.BlockSpec((2, MB, HH), lambda i: (0, i, 0)),
            pl.BlockSpec((2, MB, HH), lambda i: (0, i, 0)),
            pl.BlockSpec((H, 2 * H), lambda i: (0, 0)),
            pl.BlockSpec((1, 2 * H), lambda i: (0, 0)),
            pl.BlockSpec((2 * H, H), lambda i: (0, 0)),
            pl.BlockSpec((1, H), lambda i: (0, 0)),
        ],
        out_specs=pl.BlockSpec((2, MB, HH), lambda i: (0, i, 0)),
        out_shape=jax.ShapeDtypeStruct((2, NP, HH), jnp.float32),
    )(a, h2, agg2, w1, c1, w2, c2)


def _edge_body(h2, idxp, t01, t23, out, aggs, hbuf, accb, idxb,
               tb0, tb1, ddv, ddb, sem):
    c = lax.axis_index("c")
    s = lax.axis_index("s")
    iota = lax.broadcasted_iota(jnp.int32, (16,), 0)
    zf = jnp.zeros((16,), jnp.float32)
    trashv = jnp.full((16,), TRASH, jnp.int32)

    def _zs(i, _):
        for q in range(8):
            accb[i, pl.ds(q * 16, 16)] = zf
        return 0
    lax.fori_loop(0, 16, _zs, 0)
    ddv[pl.ds(0, 16)] = trashv

    zb = s * (SPAD // NT)

    def _zcp(i, _):
        pltpu.sync_copy(accb.at[pl.ds(0, 16)], aggs.at[pl.ds(zb + i * 16, 16)])
        return 0
    lax.fori_loop(0, (SPAD // NT) // 16, _zcp, 0)

    pltpu.sync_copy(t01.at[c], tb0)
    pltpu.sync_copy(t23.at[c], tb1)
    plsc.subcore_barrier()

    pltpu.sync_copy(idxp.at[s, 0], idxb.at[0])
    pltpu.async_copy(h2.at[c].at[idxb.at[0, pl.ds(0, EC)]], hbuf.at[0], sem)

    def _flush_block(blk):
        pltpu.sync_copy(accb.at[pl.ds(blk * 16, 16)], aggs.at[ddb.at[blk]], add=True)

    def _chunk(g, carry):
        cur = g & 1
        nxt = (g + 1) & 1

        @pl.when(g + 1 < NCH)
        def _pref():
            pltpu.sync_copy(idxp.at[s, g + 1], idxb.at[nxt])
        pltpu.make_async_copy(h2.at[c].at[idxb.at[cur, pl.ds(0, EC)]],
                              hbuf.at[cur], sem).wait()

        @pl.when(g + 1 < NCH)
        def _pref2():
            pltpu.async_copy(h2.at[c].at[idxb.at[nxt, pl.ds(0, EC)]],
                             hbuf.at[nxt], sem)

        def _group(j, gcarry):
            eb = j * 16
            dv = idxb[cur, pl.ds(EC + eb, 16)]
            i0v = idxb[cur, pl.ds(2 * EC + eb, 16)]
            i1v = idxb[cur, pl.ds(3 * EC + eb, 16)]
            cr = gcarry
            for k in range(16):
                dprev, nr = cr
                d = dv[k]
                i0 = i0v[k]
                i1 = i1v[k]
                fresh = d != dprev
                done = fresh & (dprev >= 0)

                @pl.when(done)
                def _rec():
                    dd = ddv[pl.ds(0, 16)]
                    ddv[pl.ds(0, 16)] = jnp.where(
                        iota == (nr & 15), jnp.full((16,), dprev), dd)

                @pl.when(done & ((nr & 15) == 15))
                def _fl():
                    ddb[(nr >> 4) & 1] = ddv[pl.ds(0, 16)]
                    ddv[pl.ds(0, 16)] = trashv
                    _flush_block((nr >> 4) & 1)

                nr = jnp.where(done, nr + 1, nr)
                keepv = jnp.full((16,), jnp.where(fresh, 0.0, 1.0))
                ek = eb + k
                slot = nr & 31
                for q in range(8):
                    qs = q * 16
                    hv = hbuf[cur, ek, pl.ds(qs, 16)]
                    e0 = tb0[i0, pl.ds(qs, 16)]
                    e1 = tb1[i1, pl.ds(qs, 16)]
                    m = jnp.maximum(hv + e0 + e1, 0.0)
                    a = accb[slot, pl.ds(qs, 16)]
                    accb[slot, pl.ds(qs, 16)] = a * keepv + m
                cr = (d, nr)
            return cr

        return lax.fori_loop(0, EC // 16, _group, carry)

    dprev, nr = lax.fori_loop(
        0, NCH, _chunk, (jnp.int32(-1), jnp.int32(0)))

    dd = ddv[pl.ds(0, 16)]
    ddv[pl.ds(0, 16)] = jnp.where(iota == (nr & 15), jnp.full((16,), dprev), dd)
    ddb[(nr >> 4) & 1] = ddv[pl.ds(0, 16)]
    _flush_block((nr >> 4) & 1)

    plsc.subcore_barrier()
    rb = NP // NT
    pltpu.sync_copy(aggs.at[pl.ds(s * rb, rb)], out.at[c, pl.ds(s * rb, rb)])


@jax.jit
def _edge(h2, idxp, t01, t23):
    return pl.kernel(
        _edge_body,
        out_type=jax.ShapeDtypeStruct((2, NP, HH), jnp.float32),
        mesh=plsc.VectorSubcoreMesh(core_axis_name="c", subcore_axis_name="s"),
        scratch_types=[
            pltpu.VMEM_SHARED((SPAD, HH), jnp.float32),
            pltpu.VMEM((2, EC, HH), jnp.float32),
            pltpu.VMEM((32, HH), jnp.float32),
            pltpu.VMEM((2, 6 * EC), jnp.int32),
            pltpu.VMEM((64, HH), jnp.float32),
            pltpu.VMEM((64, HH), jnp.float32),
            pltpu.VMEM((16,), jnp.int32),
            pltpu.VMEM((2, 16), jnp.int32),
            pltpu.SemaphoreType.DMA,
        ],
    )(h2, idxp, t01, t23)


def kernel(x, edge_index, edge_attr, batch, atom_emb, bond_emb, eps, W1, b1, g1, be1, W2, b2, g2, be2, Wp, bp):
    n = x.shape[0]
    src = edge_index[0]
    dst = edge_index[1]

    W1f = W1 * g1[:, None, :]
    c1 = (b1 * g1 + be1)[:, None, :]
    W2f = W2 * g2[:, None, :]
    c2 = (b2 * g2 + be2)[:, None, :]

    h = jnp.zeros((n, H), jnp.float32)
    for f in range(9):
        h = h + atom_emb[f][x[:, f]]
    hp = jnp.zeros((NP, H), jnp.float32).at[:n].set(h)
    h2 = jnp.stack([hp[:, :HH], hp[:, HH:]], 0)

    bt01 = (bond_emb[:, 0, :, None, :] + bond_emb[:, 1, None, :, :]).reshape(4, 64, H)
    bt23 = (bond_emb[:, 2, :, None, :] + bond_emb[:, 3, None, :, :]).reshape(4, 64, H)
    bt01h = jnp.stack([bt01[:, :, :HH], bt01[:, :, HH:]], 1)
    bt23h = jnp.stack([bt23[:, :, :HH], bt23[:, :, HH:]], 1)
    i01 = edge_attr[:, 0] + 8 * edge_attr[:, 1]
    i23 = edge_attr[:, 2] + 8 * edge_attr[:, 3]

    dsts, srcs, i01s, i23s = lax.sort([dst, src, i01, i23], num_keys=1)
    padn = NEP - NE
    dstp = jnp.concatenate([dsts, jnp.full((padn,), TRASH, jnp.int32)])
    srcp = jnp.concatenate([srcs, jnp.zeros((padn,), jnp.int32)])
    i01p = jnp.concatenate([i01s, jnp.zeros((padn,), jnp.int32)])
    i23p = jnp.concatenate([i23s, jnp.zeros((padn,), jnp.int32)])
    z = jnp.zeros((NEP,), jnp.int32)
    idxp = (jnp.stack([srcp, dstp, i01p, i23p, z, z], 0)
            .reshape(6, NT, NCH, EC).transpose(1, 2, 0, 3).reshape(NT, NCH, 6 * EC))

    for i in range(4):
        agg2 = _edge(h2, idxp, bt01h[i], bt23h[i])
        h2 = _mlp(1.0 + eps[i:i + 1], h2, agg2, W1f[i], c1[i], W2f[i], c2[i])

    nr = jnp.concatenate([h2[0], h2[1]], axis=1)[:n]
    sums = jax.ops.segment_sum(nr, batch, num_segments=64)
    cnt = jax.ops.segment_sum(jnp.ones((n,), jnp.float32), batch, num_segments=64)
    hg = sums / jnp.maximum(cnt, 1.0)[:, None]
    return hg @ Wp + bp
